# baseline (speedup 1.0000x reference)
import functools

import jax
import jax.numpy as jnp
from jax import lax
from jax.experimental import pallas as pl
from jax.experimental.pallas import tpu as pltpu
from jax.experimental.pallas import tpu_sc as plsc

B = 2
L = 512
D = 256
C = 5
IMG = 512
GSD = 0.2
GRID = 3
SPACING = 2
TD = 6
N = L * GRID * GRID
Q = N * C
ROW = 32
GQ = B * N

_NC, _NS = 2, 16
_NW = _NC * _NS
_PER = GQ // _NW
_CH = 96
_NCHUNK = _PER // _CH


def _gather_tokens_sc(table, flat_idx, qc):
    mesh = plsc.VectorSubcoreMesh(core_axis_name="c", subcore_axis_name="s")

    @functools.partial(
        pl.kernel,
        mesh=mesh,
        out_type=jax.ShapeDtypeStruct((GQ, ROW), jnp.float32),
        scratch_types=[
            pltpu.VMEM((_NCHUNK, _CH), jnp.int32),
            pltpu.VMEM((_PER, ROW), jnp.float32),
            pltpu.VMEM((2 * _PER,), jnp.float32),
            pltpu.SemaphoreType.DMA,
        ],
        compiler_params=pltpu.CompilerParams(use_tc_tiling_on_sc=False,
                                             needs_layout_passes=False),
    )
    def k(table_hbm, idx_hbm, qc_hbm, out_hbm, idx_v, rows_v, qc_v, sem):
        wid = lax.axis_index("s") * _NC + lax.axis_index("c")
        base = wid * _PER
        for j in range(_NCHUNK):
            pltpu.sync_copy(idx_hbm.at[pl.ds(base + j * _CH, _CH)], idx_v.at[j])
        pltpu.sync_copy(qc_hbm.at[pl.ds(2 * base, 2 * _PER)], qc_v)
        copies = []
        for j in range(_NCHUNK):
            copies.append(
                pltpu.async_copy(
                    table_hbm.at[idx_v.at[j]],
                    rows_v.at[pl.ds(j * _CH, _CH)],
                    sem,
                )
            )
        for cp in copies:
            cp.wait()
        lanes = lax.iota(jnp.int32, 16)
        cols = 30 + (lanes & 1)
        for i in range(_PER // 8):
            rows = i * 8 + (lanes >> 1)
            vals = qc_v[pl.ds(i * 16, 16)]
            plsc.store_scatter(rows_v, [rows, cols], vals)
        pltpu.sync_copy(rows_v, out_hbm.at[pl.ds(base, _PER)])

    return k(table, flat_idx, qc)


def _prep_body(latsT_ref, coordsT_ref, wkT_ref, wvT_ref, woT_ref, wqs_ref,
               kb_ref, vo_ref):
    latsT = latsT_ref[0]
    kt = jnp.dot(wkT_ref[...], latsT,
                 preferred_element_type=jnp.float32)
    vt = jnp.dot(wvT_ref[...], latsT,
                 preferred_element_type=jnp.float32)
    s = 1.0 / float(IMG * IMG)
    lp = coordsT_ref[0] / GSD + IMG / 2.0
    lpy = lp[0:1, :]
    lpx = lp[1:2, :]
    r1 = (2.0 * s) * lpy
    r2 = (2.0 * s) * lpx
    r3 = -s * (lpy * lpy + lpx * lpx)
    qk = jnp.dot(wqs_ref[...], kt,
                 preferred_element_type=jnp.float32)[0:TD]
    for c in range(C):
        pieces = []
        if c > 0:
            pieces.append(jnp.zeros((c * TD, L), jnp.float32))
        pieces.append(qk)
        rest = 30 - (c * TD + TD)
        if rest > 0:
            pieces.append(jnp.zeros((rest, L), jnp.float32))
        pieces.append(r1)
        pieces.append(r2)
        kb_ref[0, c] = jnp.concatenate(pieces, axis=0)
    vo_ref[0] = jnp.concatenate(
        [jnp.dot(woT_ref[...], vt, preferred_element_type=jnp.float32)[0:1],
         r3, jnp.zeros((6, L), jnp.float32)], axis=0)


def _prep(latsT, coordsT, WkT, WvT, WoT8, wqs):
    return pl.pallas_call(
        _prep_body,
        grid=(B,),
        in_specs=[
            pl.BlockSpec((1, D, L), lambda b: (b, 0, 0)),
            pl.BlockSpec((1, 2, L), lambda b: (b, 0, 0)),
            pl.BlockSpec((D, D), lambda b: (0, 0)),
            pl.BlockSpec((D, D), lambda b: (0, 0)),
            pl.BlockSpec((8, D), lambda b: (0, 0)),
            pl.BlockSpec((8, D), lambda b: (0, 0)),
        ],
        out_specs=[
            pl.BlockSpec((1, C, ROW, L), lambda b: (b, 0, 0, 0)),
            pl.BlockSpec((1, 8, L), lambda b: (b, 0, 0)),
        ],
        out_shape=[
            jax.ShapeDtypeStruct((B, C, ROW, L), jnp.float32),
            jax.ShapeDtypeStruct((B, 8, L), jnp.float32),
        ],
    )(latsT, coordsT, WkT, WvT, WoT8, wqs)


BP = 576
NPB = N // BP
GL = BP // 9


def _attn_body(tok_ref, kb_ref, vo_ref, g_ref, out_ref):
    tok = tok_ref[0]
    vo = vo_ref[0, 0:1, :]
    bias = vo_ref[0, 1:2, :]
    acc = jnp.zeros((BP, 1), jnp.float32)
    for c in range(C):
        logits = jnp.dot(tok, kb_ref[0, c],
                         preferred_element_type=jnp.float32) + bias
        m = jnp.max(logits, axis=-1, keepdims=True)
        e = jnp.exp(logits - m)
        den = jnp.sum(e, axis=-1, keepdims=True)
        num = jnp.sum(e * vo, axis=-1, keepdims=True)
        pred = num / den
        gt = tok[:, c * TD:c * TD + 1]
        d = pred - gt
        acc = acc + d * d
    out_ref[0, 0] = jnp.dot(g_ref[...], acc,
                            preferred_element_type=jnp.float32)


def _attn(tok32, kb, vo, g):
    return pl.pallas_call(
        _attn_body,
        grid=(B, NPB),
        in_specs=[
            pl.BlockSpec((1, BP, ROW), lambda b, qb: (b, qb, 0)),
            pl.BlockSpec((1, C, ROW, L), lambda b, qb: (b, 0, 0, 0)),
            pl.BlockSpec((1, 8, L), lambda b, qb: (b, 0, 0)),
            pl.BlockSpec((GL, BP), lambda b, qb: (0, 0)),
        ],
        out_specs=pl.BlockSpec((1, 1, GL, 1), lambda b, qb: (b, qb, 0, 0)),
        out_shape=jax.ShapeDtypeStruct((B, NPB, GL, 1), jnp.float32),
        compiler_params=pltpu.CompilerParams(
            dimension_semantics=("parallel", "parallel")),
    )(tok32, kb, vo, g)


def kernel(initial_positions, final_latents, final_coords, image_err,
           Wq, Wk, Wv, Wo):
    f32 = jnp.float32
    pos_pix = initial_positions / GSD + IMG / 2.0
    off = (jnp.arange(GRID, dtype=f32) - GRID // 2) * SPACING
    oy, ox = jnp.meshgrid(off, off, indexing="ij")
    grid_off = jnp.stack([oy.ravel(), ox.ravel()], axis=-1)
    sc = pos_pix[:, :, None, :] + grid_off[None, None, :, :]
    sc = jnp.clip(sc, 0.0, IMG - 1.0)
    sc_flat = sc.reshape(B, N, 2)
    idx = jnp.round(sc_flat).astype(jnp.int32)
    y = idx[..., 0]
    x = idx[..., 1]
    flat_idx = (jnp.arange(B, dtype=jnp.int32)[:, None] * (IMG * IMG)
                + y * IMG + x).reshape(GQ)

    imgT = jnp.transpose(image_err, (0, 2, 3, 1, 4)).reshape(B, IMG * IMG,
                                                             C * TD)
    table = jnp.pad(imgT, ((0, 0), (0, 0), (0, ROW - C * TD)))
    table = table.reshape(B * IMG * IMG, ROW)

    gathered = _gather_tokens_sc(table, flat_idx,
                                 sc_flat.reshape(2 * GQ))
    tok32 = gathered.reshape(B, N, ROW)

    wqs = jnp.pad(Wq * (1.0 / 16.0), ((0, 2), (0, 0)))
    latsT = jnp.transpose(final_latents, (0, 2, 1))
    coordsT = jnp.transpose(final_coords, (0, 2, 1))
    WkT = Wk.T
    WvT = Wv.T
    WoT8 = jnp.pad(Wo.T, ((0, 7), (0, 0)))

    kb, vo = _prep(latsT, coordsT, WkT, WvT, WoT8, wqs)

    g = jnp.repeat(jnp.eye(GL, dtype=f32), 9, axis=1) * (1.0 / 45.0)

    out = _attn(tok32, kb, vo, g)
    return out.reshape(B, L)
    out = _attn(tokx, kb, vo, g)
    return out.reshape(B, L)
